# baseline (speedup 1.0000x reference)
import functools

import jax
import jax.numpy as jnp
from jax.experimental import pallas as pl
from jax.experimental.pallas import tpu as pltpu
from jax.experimental.pallas import tpu_sc as plsc

_GATHER_WINDOW = 128


def _sc_gather(table, idx_flat):
    m = idx_flat.shape[1]
    f = table.shape[1]
    mesh = plsc.VectorSubcoreMesh(core_axis_name="core", subcore_axis_name="subcore")

    @functools.partial(
        pl.kernel,
        out_type=jax.ShapeDtypeStruct((m, f), table.dtype),
        mesh=mesh,
    )
    def gather_kernel(table_hbm, idx_hbm, out_hbm):
        def body(idx_vmem, out_vmem):
            pltpu.sync_copy(table_hbm.at[idx_vmem.at[0]], out_vmem)

        pltpu.emit_pipeline(
            body,
            grid=(m // _GATHER_WINDOW,),
            in_specs=[pl.BlockSpec((1, _GATHER_WINDOW), index_map=lambda i: (0, i))],
            out_specs=[pl.BlockSpec((_GATHER_WINDOW, f), index_map=lambda i: (i, 0))],
            core_axis_name=("core", "subcore"),
            dimension_semantics=(pltpu.PARALLEL,),
        )(idx_hbm, out_hbm)

    return gather_kernel(table, idx_flat)


def _add_body(x_ref, d_ref, o_ref):
    f = x_ref.shape[-1]
    o_ref[...] = x_ref[...] + d_ref[..., :f]


def _tc_broadcast_add(x, demb4):
    b, t, n, f = x.shape
    nb = 0
    for cand in range(1024, 0, -8):
        if n % cand == 0:
            nb = cand
            break
    assert nb > 0, n
    fp = demb4.shape[-1]
    grid = (b, n // nb)
    return pl.pallas_call(
        _add_body,
        grid=grid,
        in_specs=[
            pl.BlockSpec((1, t, nb, f), lambda i, j: (i, 0, j, 0)),
            pl.BlockSpec((1, 1, nb, fp), lambda i, j: (i, 0, j, 0)),
        ],
        out_specs=pl.BlockSpec((1, t, nb, f), lambda i, j: (i, 0, j, 0)),
        out_shape=jax.ShapeDtypeStruct(x.shape, x.dtype),
    )(x, demb4)


def _flat_add_body(x_ref, d_ref, o_ref):
    o_ref[...] = x_ref[...] + d_ref[...]


def _tc_flat_add(x3, demb3):
    b, t, nf = x3.shape
    ch = 0
    for cand in range(32768, 0, -128):
        if nf % cand == 0:
            ch = cand
            break
    grid = (b, nf // ch)
    return pl.pallas_call(
        _flat_add_body,
        grid=grid,
        in_specs=[
            pl.BlockSpec((1, t, ch), lambda i, c: (i, 0, c)),
            pl.BlockSpec((1, 1, ch), lambda i, c: (i, 0, c)),
        ],
        out_specs=pl.BlockSpec((1, t, ch), lambda i, c: (i, 0, c)),
        out_shape=jax.ShapeDtypeStruct(x3.shape, x3.dtype),
    )(x3, demb3)


def kernel(degree, x, emb_table):
    b, t, n, f = x.shape
    m = b * n
    demb = jnp.broadcast_to(emb_table[:1, :], (m, f))
    x3 = x.reshape(b, t, n * f)
    demb3 = demb.reshape(b, 1, n * f)
    return _tc_flat_add(x3, demb3).reshape(x.shape)


# speedup vs baseline: 1.9314x; 1.9314x over previous
import functools

import jax
import jax.numpy as jnp
from jax.experimental import pallas as pl
from jax.experimental.pallas import tpu as pltpu
from jax.experimental.pallas import tpu_sc as plsc

_GATHER_WINDOW = 128


def _sc_gather(table, idx_flat):
    m = idx_flat.shape[1]
    f = table.shape[1]
    mesh = plsc.VectorSubcoreMesh(core_axis_name="core", subcore_axis_name="subcore")

    @functools.partial(
        pl.kernel,
        out_type=jax.ShapeDtypeStruct((m, f), table.dtype),
        mesh=mesh,
    )
    def gather_kernel(table_hbm, idx_hbm, out_hbm):
        def body(idx_vmem, out_vmem):
            pltpu.sync_copy(table_hbm.at[idx_vmem.at[0]], out_vmem)

        pltpu.emit_pipeline(
            body,
            grid=(m // _GATHER_WINDOW,),
            in_specs=[pl.BlockSpec((1, _GATHER_WINDOW), index_map=lambda i: (0, i))],
            out_specs=[pl.BlockSpec((_GATHER_WINDOW, f), index_map=lambda i: (i, 0))],
            core_axis_name=("core", "subcore"),
            dimension_semantics=(pltpu.PARALLEL,),
        )(idx_hbm, out_hbm)

    return gather_kernel(table, idx_flat)


def _add_body(x_ref, d_ref, o_ref):
    f = x_ref.shape[-1]
    o_ref[...] = x_ref[...] + d_ref[..., :f]


def _tc_broadcast_add(x2, demb3, t):
    bt, n, f = x2.shape
    fp = demb3.shape[-1]
    rows = 2
    assert bt % rows == 0 and t % rows == 0
    grid = (bt // rows,)
    return pl.pallas_call(
        _add_body,
        grid=grid,
        in_specs=[
            pl.BlockSpec((rows, n, f), lambda i: (i, 0, 0)),
            pl.BlockSpec((1, n, fp), lambda i: (i // (t // rows), 0, 0)),
        ],
        out_specs=pl.BlockSpec((rows, n, f), lambda i: (i, 0, 0)),
        out_shape=jax.ShapeDtypeStruct(x2.shape, x2.dtype),
    )(x2, demb3)


def kernel(degree, x, emb_table):
    b, t, n, f = x.shape
    m = b * n
    fp = 128
    table_pad = jnp.pad(emb_table, ((0, 0), (0, fp - f)))
    idx = degree.astype(jnp.int32).reshape(1, m)
    pad = (-m) % _GATHER_WINDOW
    if pad:
        idx = jnp.pad(idx, ((0, 0), (0, pad)))
    demb = _sc_gather(table_pad, idx)
    if pad:
        demb = demb[:m]
    demb3 = demb.reshape(b, n, fp)
    x2 = x.reshape(b * t, n, f)
    out2 = _tc_broadcast_add(x2, demb3, t)
    return out2.reshape(x.shape)
